# Initial kernel scaffold; baseline (speedup 1.0000x reference)
import jax
import jax.numpy as jnp
from jax.experimental import pallas as pl


def kernel(x, edge_index, W_gcn, b_gcn, W_fc1, b_fc1, W_fc2, b_fc2, W_out, b_out):
    raise NotImplementedError("write your pallas kernel here")


# speedup vs baseline: 6.9367x; 6.9367x over previous
import functools

import jax
import jax.numpy as jnp
from jax import lax
from jax.experimental import pallas as pl
from jax.experimental.pallas import tpu as pltpu
from jax.experimental.pallas import tpu_sc as plsc

NC = 2
NS = 16
NW = NC * NS
K = 80
DEGW = 16


def _sc_deg(np_, nch):
    mesh = plsc.VectorSubcoreMesh(core_axis_name="c", subcore_axis_name="s")
    rows = np_ // NS

    @functools.partial(
        pl.kernel,
        out_type=jax.ShapeDtypeStruct((2 * np_, DEGW), jnp.float32),
        mesh=mesh,
        scratch_types=[
            pltpu.VMEM_SHARED((np_, DEGW), jnp.float32),
            pltpu.VMEM((nch, K), jnp.int32),
            pltpu.VMEM((K, DEGW), jnp.float32),
        ],
    )
    def k(dst_hbm, zeros_hbm, ones_hbm, out_hbm, deg_sp, idx_v, ones_v):
        cid = lax.axis_index("c")
        sid = lax.axis_index("s")
        wid = sid * NC + cid
        pltpu.sync_copy(zeros_hbm.at[pl.ds(sid * rows, rows)],
                        deg_sp.at[pl.ds(sid * rows, rows)])
        pltpu.sync_copy(dst_hbm.at[wid], idx_v)
        pltpu.sync_copy(ones_hbm, ones_v)
        plsc.subcore_barrier()

        def body(j, c):
            pltpu.sync_copy(ones_v, deg_sp.at[idx_v.at[j]], add=True)
            return c

        lax.fori_loop(0, nch, body, 0)
        plsc.subcore_barrier()
        pltpu.sync_copy(deg_sp.at[pl.ds(sid * rows, rows)],
                        out_hbm.at[pl.ds(cid * np_ + sid * rows, rows)])

    return k


def _sc_agg(np_, d, nch):
    mesh = plsc.VectorSubcoreMesh(core_axis_name="c", subcore_axis_name="s")
    rows = np_ // NS

    @functools.partial(
        pl.kernel,
        out_type=jax.ShapeDtypeStruct((2 * np_, d), jnp.float32),
        mesh=mesh,
        scratch_types=[
            pltpu.VMEM_SHARED((np_, d), jnp.float32),
            pltpu.VMEM((nch, K), jnp.int32),
            pltpu.VMEM((nch, K), jnp.int32),
            pltpu.VMEM((K, d), jnp.float32),
        ],
    )
    def k(src_hbm, dst_hbm, hn_hbm, zeros_hbm, out_hbm,
          agg_sp, sidx_v, didx_v, rows_v):
        cid = lax.axis_index("c")
        sid = lax.axis_index("s")
        wid = sid * NC + cid
        pltpu.sync_copy(zeros_hbm.at[pl.ds(sid * rows, rows)],
                        agg_sp.at[pl.ds(sid * rows, rows)])
        pltpu.sync_copy(src_hbm.at[wid], sidx_v)
        pltpu.sync_copy(dst_hbm.at[wid], didx_v)
        plsc.subcore_barrier()

        def body(j, c):
            pltpu.sync_copy(hn_hbm.at[sidx_v.at[j]], rows_v)
            pltpu.sync_copy(rows_v, agg_sp.at[didx_v.at[j]], add=True)
            return c

        lax.fori_loop(0, nch, body, 0)
        plsc.subcore_barrier()
        pltpu.sync_copy(agg_sp.at[pl.ds(sid * rows, rows)],
                        out_hbm.at[pl.ds(cid * np_ + sid * rows, rows)])

    return k


def _sc_ey(e, d, nch, ew):
    mesh = plsc.VectorSubcoreMesh(core_axis_name="c", subcore_axis_name="s")

    @functools.partial(
        pl.kernel,
        out_type=jax.ShapeDtypeStruct((e, d), jnp.float32),
        mesh=mesh,
        scratch_types=[
            pltpu.VMEM((nch, K), jnp.int32),
            pltpu.VMEM((nch, K), jnp.int32),
            pltpu.VMEM((K, d), jnp.float32),
        ],
    )
    def k(src_hbm, dst_hbm, y_hbm, out_hbm, sidx_v, didx_v, buf_v):
        cid = lax.axis_index("c")
        sid = lax.axis_index("s")
        wid = sid * NC + cid
        pltpu.sync_copy(src_hbm.at[wid], sidx_v)
        pltpu.sync_copy(dst_hbm.at[wid], didx_v)

        def body(j, c):
            pltpu.sync_copy(y_hbm.at[sidx_v.at[j]], buf_v)
            pltpu.sync_copy(y_hbm.at[didx_v.at[j]], buf_v, add=True)
            pltpu.sync_copy(buf_v, out_hbm.at[pl.ds(wid * ew + j * K, K)])
            return c

        lax.fori_loop(0, nch, body, 0)

    return k


_DOT = functools.partial(jnp.dot, precision=jax.lax.Precision.HIGHEST)


def _tc_hn_body(x_ref, w_ref, d0_ref, d1_ref, hn_ref):
    dinv = lax.rsqrt(d0_ref[...] + d1_ref[...] + 1.0)
    hn_ref[...] = _DOT(x_ref[...], w_ref[...]) * dinv


def _tc_nodes_body(a0_ref, a1_ref, hn_ref, d0_ref, d1_ref, bg_ref,
                   w1_ref, b1_ref, w2_ref, b2_ref, y_ref):
    dinv = lax.rsqrt(d0_ref[...] + d1_ref[...] + 1.0)
    hn = hn_ref[...]
    g = dinv * (a0_ref[...] + a1_ref[...] + hn) + bg_ref[...]
    t = jnp.maximum(g, 0.0)
    t = jnp.maximum(_DOT(t, w1_ref[...]) + b1_ref[...], 0.0)
    xh = jnp.maximum(_DOT(t, w2_ref[...]) + b2_ref[...], 0.0)
    y_ref[...] = _DOT(xh, w1_ref[...])


def _tc_edges_body(ey_ref, b1_ref, w2_ref, b2_ref, wo_ref, bo_ref, out_ref):
    e1 = jnp.maximum(ey_ref[...] * 0.5 + b1_ref[...], 0.0)
    e2 = jnp.maximum(_DOT(e1, w2_ref[...]) + b2_ref[...], 0.0)
    s = _DOT(e2, wo_ref[...]) + bo_ref[...]
    m = jnp.max(s, axis=1, keepdims=True)
    out_ref[...] = s - (m + jnp.log(jnp.sum(jnp.exp(s - m), axis=1,
                                            keepdims=True)))


def kernel(x, edge_index, W_gcn, b_gcn, W_fc1, b_fc1, W_fc2, b_fc2,
           W_out, b_out):
    n, d = x.shape
    e = edge_index.shape[1]
    ew = e // NW
    nch = ew // K
    np_ = ((n + 127) // 128) * 128

    src3 = edge_index[0].reshape(NW, nch, K)
    dst3 = edge_index[1].reshape(NW, nch, K)
    x_pad = jnp.pad(x, ((0, np_ - n), (0, 0)))
    zeros1 = jnp.zeros((np_, DEGW), jnp.float32)
    zeros2 = jnp.zeros((np_, d), jnp.float32)
    ones_k = jnp.ones((K, DEGW), jnp.float32)

    degs = _sc_deg(np_, nch)(dst3, zeros1, ones_k)
    d0 = degs[:np_, 0:1]
    d1 = degs[np_:, 0:1]

    hn = pl.pallas_call(
        _tc_hn_body,
        out_shape=jax.ShapeDtypeStruct((np_, d), jnp.float32),
    )(x_pad, W_gcn, d0, d1)

    aggs = _sc_agg(np_, d, nch)(src3, dst3, hn, zeros2)

    specs = [
        pl.BlockSpec((np_, d), lambda i: (0, 0)),
        pl.BlockSpec((np_, d), lambda i: (1, 0)),
        pl.BlockSpec((np_, d), lambda i: (0, 0)),
        pl.BlockSpec((np_, 1), lambda i: (0, 0)),
        pl.BlockSpec((np_, 1), lambda i: (0, 0)),
        pl.BlockSpec((1, d), lambda i: (0, 0)),
        pl.BlockSpec((d, d), lambda i: (0, 0)),
        pl.BlockSpec((1, d), lambda i: (0, 0)),
        pl.BlockSpec((d, d), lambda i: (0, 0)),
        pl.BlockSpec((1, d), lambda i: (0, 0)),
    ]
    y = pl.pallas_call(
        _tc_nodes_body,
        out_shape=jax.ShapeDtypeStruct((np_, d), jnp.float32),
        grid=(1,),
        in_specs=specs,
        out_specs=pl.BlockSpec((np_, d), lambda i: (0, 0)),
    )(aggs, aggs, hn, d0, d1, b_gcn.reshape(1, d), W_fc1,
      b_fc1.reshape(1, d), W_fc2, b_fc2.reshape(1, d))

    ey = _sc_ey(e, d, nch, ew)(src3, dst3, y)

    dout = W_out.shape[1]
    blk = 4000
    grid = e // blk
    out = pl.pallas_call(
        _tc_edges_body,
        out_shape=jax.ShapeDtypeStruct((e, dout), jnp.float32),
        grid=(grid,),
        in_specs=[
            pl.BlockSpec((blk, d), lambda i: (i, 0)),
            pl.BlockSpec((1, d), lambda i: (0, 0)),
            pl.BlockSpec((d, d), lambda i: (0, 0)),
            pl.BlockSpec((1, d), lambda i: (0, 0)),
            pl.BlockSpec((d, dout), lambda i: (0, 0)),
            pl.BlockSpec((1, dout), lambda i: (0, 0)),
        ],
        out_specs=pl.BlockSpec((blk, dout), lambda i: (i, 0)),
    )(ey, b_fc1.reshape(1, d), W_fc2, b_fc2.reshape(1, d), W_out,
      b_out.reshape(1, dout))
    return out


# speedup vs baseline: 9.9886x; 1.4399x over previous
import functools

import jax
import jax.numpy as jnp
from jax import lax
from jax.experimental import pallas as pl
from jax.experimental.pallas import tpu as pltpu
from jax.experimental.pallas import tpu_sc as plsc

NC = 2
NS = 16
NW = NC * NS
K = 80
DEGW = 16


def _sc_deg(np_, nch):
    mesh = plsc.VectorSubcoreMesh(core_axis_name="c", subcore_axis_name="s")
    rows = np_ // NS

    @functools.partial(
        pl.kernel,
        out_type=jax.ShapeDtypeStruct((2 * np_, DEGW), jnp.float32),
        mesh=mesh,
        scratch_types=[
            pltpu.VMEM_SHARED((np_, DEGW), jnp.float32),
            pltpu.VMEM((nch, K), jnp.int32),
            pltpu.VMEM((K, DEGW), jnp.float32),
        ],
    )
    def k(dst_hbm, zeros_hbm, ones_hbm, out_hbm, deg_sp, idx_v, ones_v):
        cid = lax.axis_index("c")
        sid = lax.axis_index("s")
        wid = sid * NC + cid
        pltpu.sync_copy(zeros_hbm.at[pl.ds(sid * rows, rows)],
                        deg_sp.at[pl.ds(sid * rows, rows)])
        pltpu.sync_copy(dst_hbm.at[wid], idx_v)
        pltpu.sync_copy(ones_hbm, ones_v)
        plsc.subcore_barrier()

        def body(j, c):
            pltpu.sync_copy(ones_v, deg_sp.at[idx_v.at[j]], add=True)
            return c

        lax.fori_loop(0, nch, body, 0)
        plsc.subcore_barrier()
        pltpu.sync_copy(deg_sp.at[pl.ds(sid * rows, rows)],
                        out_hbm.at[pl.ds(cid * np_ + sid * rows, rows)])

    return k


def _sc_agg(np_, d, nch):
    mesh = plsc.VectorSubcoreMesh(core_axis_name="c", subcore_axis_name="s")
    rows = np_ // NS

    @functools.partial(
        pl.kernel,
        out_type=jax.ShapeDtypeStruct((2 * np_, d), jnp.float32),
        mesh=mesh,
        scratch_types=[
            pltpu.VMEM_SHARED((np_, d), jnp.float32),
            pltpu.VMEM((nch, K), jnp.int32),
            pltpu.VMEM((nch, K), jnp.int32),
            pltpu.VMEM((K, d), jnp.float32),
        ],
    )
    def k(src_hbm, dst_hbm, hn_hbm, zeros_hbm, out_hbm,
          agg_sp, sidx_v, didx_v, rows_v):
        cid = lax.axis_index("c")
        sid = lax.axis_index("s")
        wid = sid * NC + cid
        pltpu.sync_copy(zeros_hbm.at[pl.ds(sid * rows, rows)],
                        agg_sp.at[pl.ds(sid * rows, rows)])
        pltpu.sync_copy(src_hbm.at[wid], sidx_v)
        pltpu.sync_copy(dst_hbm.at[wid], didx_v)
        plsc.subcore_barrier()

        def body(j, c):
            pltpu.sync_copy(hn_hbm.at[sidx_v.at[j]], rows_v)
            pltpu.sync_copy(rows_v, agg_sp.at[didx_v.at[j]], add=True)
            return c

        lax.fori_loop(0, nch, body, 0)
        plsc.subcore_barrier()
        pltpu.sync_copy(agg_sp.at[pl.ds(sid * rows, rows)],
                        out_hbm.at[pl.ds(cid * np_ + sid * rows, rows)])

    return k


def _sc_ey(e, d, nch, ew):
    mesh = plsc.VectorSubcoreMesh(core_axis_name="c", subcore_axis_name="s")

    @functools.partial(
        pl.kernel,
        out_type=jax.ShapeDtypeStruct((e, d), jnp.float32),
        mesh=mesh,
        scratch_types=[
            pltpu.VMEM((nch, K), jnp.int32),
            pltpu.VMEM((nch, K), jnp.int32),
            pltpu.VMEM((K, d), jnp.float32),
        ],
    )
    def k(src_hbm, dst_hbm, y_hbm, out_hbm, sidx_v, didx_v, buf_v):
        cid = lax.axis_index("c")
        sid = lax.axis_index("s")
        wid = sid * NC + cid
        pltpu.sync_copy(src_hbm.at[wid], sidx_v)
        pltpu.sync_copy(dst_hbm.at[wid], didx_v)

        def body(j, c):
            pltpu.sync_copy(y_hbm.at[sidx_v.at[j]], buf_v)
            pltpu.sync_copy(y_hbm.at[didx_v.at[j]], buf_v, add=True)
            pltpu.sync_copy(buf_v, out_hbm.at[pl.ds(wid * ew + j * K, K)])
            return c

        lax.fori_loop(0, nch, body, 0)

    return k


_DOT = functools.partial(jnp.dot, precision=jax.lax.Precision.DEFAULT)


def _tc_hn_body(x_ref, w_ref, d0_ref, d1_ref, hn_ref):
    dinv = lax.rsqrt(d0_ref[...] + d1_ref[...] + 1.0)
    hn_ref[...] = _DOT(x_ref[...], w_ref[...]) * dinv


def _tc_nodes_body(a0_ref, a1_ref, hn_ref, d0_ref, d1_ref, bg_ref,
                   w1_ref, b1_ref, w2_ref, b2_ref, y_ref):
    dinv = lax.rsqrt(d0_ref[...] + d1_ref[...] + 1.0)
    hn = hn_ref[...]
    g = dinv * (a0_ref[...] + a1_ref[...] + hn) + bg_ref[...]
    t = jnp.maximum(g, 0.0)
    t = jnp.maximum(_DOT(t, w1_ref[...]) + b1_ref[...], 0.0)
    xh = jnp.maximum(_DOT(t, w2_ref[...]) + b2_ref[...], 0.0)
    y_ref[...] = _DOT(xh, w1_ref[...])


def _tc_edges_body(ey_ref, b1_ref, w2_ref, b2_ref, wo_ref, bo_ref, out_ref):
    e1 = jnp.maximum(ey_ref[...] * 0.5 + b1_ref[...], 0.0)
    e2 = jnp.maximum(_DOT(e1, w2_ref[...]) + b2_ref[...], 0.0)
    s = _DOT(e2, wo_ref[...]) + bo_ref[...]
    m = jnp.max(s, axis=1, keepdims=True)
    out_ref[...] = s - (m + jnp.log(jnp.sum(jnp.exp(s - m), axis=1,
                                            keepdims=True)))


def kernel(x, edge_index, W_gcn, b_gcn, W_fc1, b_fc1, W_fc2, b_fc2,
           W_out, b_out):
    n, d = x.shape
    e = edge_index.shape[1]
    ew = e // NW
    nch = ew // K
    np_ = ((n + 127) // 128) * 128

    src3 = edge_index[0].reshape(NW, nch, K)
    dst3 = edge_index[1].reshape(NW, nch, K)
    x_pad = jnp.pad(x, ((0, np_ - n), (0, 0)))
    zeros1 = jnp.zeros((np_, DEGW), jnp.float32)
    zeros2 = jnp.zeros((np_, d), jnp.float32)
    ones_k = jnp.ones((K, DEGW), jnp.float32)

    degs = _sc_deg(np_, nch)(dst3, zeros1, ones_k)
    d0 = degs[:np_, 0:1]
    d1 = degs[np_:, 0:1]

    hn = pl.pallas_call(
        _tc_hn_body,
        out_shape=jax.ShapeDtypeStruct((np_, d), jnp.float32),
    )(x_pad, W_gcn, d0, d1)

    aggs = _sc_agg(np_, d, nch)(src3, dst3, hn, zeros2)

    specs = [
        pl.BlockSpec((np_, d), lambda i: (0, 0)),
        pl.BlockSpec((np_, d), lambda i: (1, 0)),
        pl.BlockSpec((np_, d), lambda i: (0, 0)),
        pl.BlockSpec((np_, 1), lambda i: (0, 0)),
        pl.BlockSpec((np_, 1), lambda i: (0, 0)),
        pl.BlockSpec((1, d), lambda i: (0, 0)),
        pl.BlockSpec((d, d), lambda i: (0, 0)),
        pl.BlockSpec((1, d), lambda i: (0, 0)),
        pl.BlockSpec((d, d), lambda i: (0, 0)),
        pl.BlockSpec((1, d), lambda i: (0, 0)),
    ]
    y = pl.pallas_call(
        _tc_nodes_body,
        out_shape=jax.ShapeDtypeStruct((np_, d), jnp.float32),
        grid=(1,),
        in_specs=specs,
        out_specs=pl.BlockSpec((np_, d), lambda i: (0, 0)),
    )(aggs, aggs, hn, d0, d1, b_gcn.reshape(1, d), W_fc1,
      b_fc1.reshape(1, d), W_fc2, b_fc2.reshape(1, d))

    ey = _sc_ey(e, d, nch, ew)(src3, dst3, y)

    dout = W_out.shape[1]
    blk = 4000
    grid = e // blk
    out = pl.pallas_call(
        _tc_edges_body,
        out_shape=jax.ShapeDtypeStruct((e, dout), jnp.float32),
        grid=(grid,),
        in_specs=[
            pl.BlockSpec((blk, d), lambda i: (i, 0)),
            pl.BlockSpec((1, d), lambda i: (0, 0)),
            pl.BlockSpec((d, d), lambda i: (0, 0)),
            pl.BlockSpec((1, d), lambda i: (0, 0)),
            pl.BlockSpec((d, dout), lambda i: (0, 0)),
            pl.BlockSpec((1, dout), lambda i: (0, 0)),
        ],
        out_specs=pl.BlockSpec((blk, dout), lambda i: (i, 0)),
    )(ey, b_fc1.reshape(1, d), W_fc2, b_fc2.reshape(1, d), W_out,
      b_out.reshape(1, dout))
    return out


# speedup vs baseline: 12.6115x; 1.2626x over previous
import functools

import jax
import jax.numpy as jnp
from jax import lax
from jax.experimental import pallas as pl
from jax.experimental.pallas import tpu as pltpu
from jax.experimental.pallas import tpu_sc as plsc

NC = 2
NS = 16
NW = NC * NS
K = 80
DEGW = 16


def _sc_deg(n, nch):
    mesh = plsc.VectorSubcoreMesh(core_axis_name="c", subcore_axis_name="s")
    rows = n // NS
    G = 5
    ngrp = nch // G
    assert nch % G == 0

    @functools.partial(
        pl.kernel,
        out_type=jax.ShapeDtypeStruct((2 * n, DEGW), jnp.float32),
        mesh=mesh,
        scratch_types=[
            pltpu.VMEM_SHARED((n, DEGW), jnp.float32),
            pltpu.VMEM((nch, K), jnp.int32),
            pltpu.VMEM((K, DEGW), jnp.float32),
            pltpu.SemaphoreType.DMA,
        ],
    )
    def k(dst_hbm, zeros_hbm, ones_hbm, out_hbm, deg_sp, idx_v, ones_v, sem):
        cid = lax.axis_index("c")
        sid = lax.axis_index("s")
        wid = sid * NC + cid
        pltpu.sync_copy(zeros_hbm, deg_sp.at[pl.ds(sid * rows, rows)])
        pltpu.sync_copy(dst_hbm.at[wid], idx_v)
        pltpu.sync_copy(ones_hbm, ones_v)
        plsc.subcore_barrier()

        def fire(t):
            for u in range(G):
                pltpu.async_copy(ones_v, deg_sp.at[idx_v.at[t * G + u]], sem,
                                 add=True)

        def drain(t):
            for u in range(G):
                pltpu.make_async_copy(ones_v, deg_sp.at[idx_v.at[t * G + u]],
                                      sem).wait()

        def body(t, c):
            fire(t)

            @pl.when(t > 0)
            def _():
                drain(t - 1)

            return c

        lax.fori_loop(0, ngrp, body, 0)
        drain(ngrp - 1)
        plsc.subcore_barrier()
        pltpu.sync_copy(deg_sp.at[pl.ds(sid * rows, rows)],
                        out_hbm.at[pl.ds(cid * n + sid * rows, rows)])

    return k


def _sc_agg(n, d, nch, ew):
    mesh = plsc.VectorSubcoreMesh(core_axis_name="c", subcore_axis_name="s")
    rows = n // NS
    T = nch // 2
    assert nch % 2 == 1

    @functools.partial(
        pl.kernel,
        out_type=jax.ShapeDtypeStruct((2 * n, d), jnp.float32),
        mesh=mesh,
        scratch_types=[
            pltpu.VMEM_SHARED((n, d), jnp.float32),
            pltpu.VMEM((ew,), jnp.int32),
            pltpu.VMEM((nch, K), jnp.int32),
            pltpu.VMEM((K, d), jnp.float32),
            pltpu.VMEM((K, d), jnp.float32),
            pltpu.SemaphoreType.DMA,
            pltpu.SemaphoreType.DMA,
            pltpu.SemaphoreType.DMA,
            pltpu.SemaphoreType.DMA,
        ],
    )
    def k(src_hbm, dst_hbm, hn_hbm, zeros_hbm, out_hbm,
          agg_sp, sidx_v, didx_v, buf0, buf1, sa0, sa1, sb0, sb1):
        cid = lax.axis_index("c")
        sid = lax.axis_index("s")
        wid = sid * NC + cid
        pltpu.sync_copy(zeros_hbm, agg_sp.at[pl.ds(sid * rows, rows)])
        pltpu.sync_copy(src_hbm.at[wid], sidx_v)
        pltpu.sync_copy(dst_hbm.at[wid], didx_v)
        plsc.subcore_barrier()

        bufs = (buf0, buf1)
        sa = (sa0, sa1)
        sb = (sb0, sb1)

        def ga(j, b):
            pltpu.async_copy(hn_hbm.at[sidx_v.at[pl.ds(j * K, K)]], bufs[b],
                             sa[b])

        def wa(j, b):
            pltpu.make_async_copy(hn_hbm.at[sidx_v.at[pl.ds(j * K, K)]],
                                  bufs[b], sa[b]).wait()

        def gb(j, b):
            pltpu.async_copy(bufs[b], agg_sp.at[didx_v.at[j]], sb[b], add=True)

        def wb(j, b):
            pltpu.make_async_copy(bufs[b], agg_sp.at[didx_v.at[j]],
                                  sb[b]).wait()

        ga(0, 0)

        def body(t, c):
            j0 = 2 * t
            wa(j0, 0)

            @pl.when(t > 0)
            def _():
                wb(j0 - 1, 1)

            ga(j0 + 1, 1)
            gb(j0, 0)
            wa(j0 + 1, 1)
            wb(j0, 0)
            ga(j0 + 2, 0)
            gb(j0 + 1, 1)
            return c

        lax.fori_loop(0, T, body, 0)
        wb(nch - 2, 1)
        wa(nch - 1, 0)
        gb(nch - 1, 0)
        wb(nch - 1, 0)
        plsc.subcore_barrier()
        pltpu.sync_copy(agg_sp.at[pl.ds(sid * rows, rows)],
                        out_hbm.at[pl.ds(cid * n + sid * rows, rows)])

    return k


def _sc_ey(e, d, nch, ew):
    mesh = plsc.VectorSubcoreMesh(core_axis_name="c", subcore_axis_name="s")
    T = nch // 2
    assert nch % 2 == 1

    @functools.partial(
        pl.kernel,
        out_type=jax.ShapeDtypeStruct((e, d), jnp.float32),
        mesh=mesh,
        scratch_types=[
            pltpu.VMEM((ew,), jnp.int32),
            pltpu.VMEM((ew,), jnp.int32),
            pltpu.VMEM((K, d), jnp.float32),
            pltpu.VMEM((K, d), jnp.float32),
            pltpu.SemaphoreType.DMA,
            pltpu.SemaphoreType.DMA,
            pltpu.SemaphoreType.DMA,
            pltpu.SemaphoreType.DMA,
            pltpu.SemaphoreType.DMA,
            pltpu.SemaphoreType.DMA,
        ],
    )
    def k(src_hbm, dst_hbm, y_hbm, out_hbm, sidx_v, didx_v, buf0, buf1,
          sa0, sa1, sb0, sb1, sc0, sc1):
        cid = lax.axis_index("c")
        sid = lax.axis_index("s")
        wid = sid * NC + cid
        base = wid * ew
        pltpu.sync_copy(src_hbm.at[wid], sidx_v)
        pltpu.sync_copy(dst_hbm.at[wid], didx_v)

        bufs = (buf0, buf1)
        sa = (sa0, sa1)
        sb = (sb0, sb1)
        sc = (sc0, sc1)

        def ga(j, b):
            pltpu.async_copy(y_hbm.at[sidx_v.at[pl.ds(j * K, K)]], bufs[b],
                             sa[b])

        def wa(j, b):
            pltpu.make_async_copy(y_hbm.at[sidx_v.at[pl.ds(j * K, K)]],
                                  bufs[b], sa[b]).wait()

        def gb(j, b):
            pltpu.async_copy(y_hbm.at[didx_v.at[pl.ds(j * K, K)]], bufs[b],
                             sb[b], add=True)

        def wb(j, b):
            pltpu.make_async_copy(y_hbm.at[didx_v.at[pl.ds(j * K, K)]],
                                  bufs[b], sb[b]).wait()

        def gc(j, b):
            pltpu.async_copy(bufs[b], out_hbm.at[pl.ds(base + j * K, K)],
                             sc[b])

        def wc(j, b):
            pltpu.make_async_copy(bufs[b], out_hbm.at[pl.ds(base + j * K, K)],
                                  sc[b]).wait()

        ga(0, 0)

        def body(t, c):
            j0 = 2 * t
            wa(j0, 0)
            gb(j0, 0)

            @pl.when(t > 0)
            def _():
                wc(j0 - 1, 1)

            ga(j0 + 1, 1)
            wb(j0, 0)
            gc(j0, 0)
            wa(j0 + 1, 1)
            gb(j0 + 1, 1)
            wc(j0, 0)
            ga(j0 + 2, 0)
            wb(j0 + 1, 1)
            gc(j0 + 1, 1)
            return c

        lax.fori_loop(0, T, body, 0)
        wc(nch - 2, 1)
        wa(nch - 1, 0)
        gb(nch - 1, 0)
        wb(nch - 1, 0)
        gc(nch - 1, 0)
        wc(nch - 1, 0)

    return k


_DOT = functools.partial(jnp.dot, precision=jax.lax.Precision.DEFAULT)


def _tc_h_body(x_ref, w_ref, h_ref):
    h_ref[...] = _DOT(x_ref[...], w_ref[...])


def _tc_hn_body(h_ref, d0_ref, d1_ref, hn_ref):
    dinv = lax.rsqrt(d0_ref[...] + d1_ref[...] + 1.0)
    hn_ref[...] = h_ref[...] * dinv


def _tc_nodes_body(a0_ref, a1_ref, hn_ref, d0_ref, d1_ref, bg_ref,
                   w1_ref, b1_ref, w2_ref, b2_ref, y_ref):
    dinv = lax.rsqrt(d0_ref[...] + d1_ref[...] + 1.0)
    hn = hn_ref[...]
    g = dinv * (a0_ref[...] + a1_ref[...] + hn) + bg_ref[...]
    t = jnp.maximum(g, 0.0)
    t = jnp.maximum(_DOT(t, w1_ref[...]) + b1_ref[...], 0.0)
    xh = jnp.maximum(_DOT(t, w2_ref[...]) + b2_ref[...], 0.0)
    y_ref[...] = _DOT(xh, w1_ref[...])


def _tc_edges_body(ey_ref, b1_ref, w2_ref, b2_ref, wo_ref, bo_ref, out_ref):
    e1 = jnp.maximum(ey_ref[...] * 0.5 + b1_ref[...], 0.0)
    e2 = jnp.maximum(_DOT(e1, w2_ref[...]) + b2_ref[...], 0.0)
    s = _DOT(e2, wo_ref[...]) + bo_ref[...]
    m = jnp.max(s, axis=1, keepdims=True)
    out_ref[...] = s - (m + jnp.log(jnp.sum(jnp.exp(s - m), axis=1,
                                            keepdims=True)))


def kernel(x, edge_index, W_gcn, b_gcn, W_fc1, b_fc1, W_fc2, b_fc2,
           W_out, b_out):
    n, d = x.shape
    e = edge_index.shape[1]
    ew = e // NW
    nch = ew // K
    np_ = ((n + 127) // 128) * 128

    src2 = edge_index[0].reshape(NW, ew)
    dst2 = edge_index[1].reshape(NW, ew)
    dst3 = edge_index[1].reshape(NW, nch, K)
    x_pad = jnp.pad(x, ((0, np_ - n), (0, 0)))
    rows = np_ // NS
    zeros_deg = jnp.zeros((rows, DEGW), jnp.float32)
    zeros_agg = jnp.zeros((rows, d), jnp.float32)
    ones_k = jnp.ones((K, DEGW), jnp.float32)

    h = pl.pallas_call(
        _tc_h_body,
        out_shape=jax.ShapeDtypeStruct((np_, d), jnp.float32),
    )(x_pad, W_gcn)

    degs = _sc_deg(np_, nch)(dst3, zeros_deg, ones_k)
    d0 = degs[:np_, 0:1]
    d1 = degs[np_:, 0:1]

    hn = pl.pallas_call(
        _tc_hn_body,
        out_shape=jax.ShapeDtypeStruct((np_, d), jnp.float32),
    )(h, d0, d1)

    aggs = _sc_agg(np_, d, nch, ew)(src2, dst3, hn, zeros_agg)

    specs = [
        pl.BlockSpec((np_, d), lambda i: (0, 0)),
        pl.BlockSpec((np_, d), lambda i: (1, 0)),
        pl.BlockSpec((np_, d), lambda i: (0, 0)),
        pl.BlockSpec((np_, 1), lambda i: (0, 0)),
        pl.BlockSpec((np_, 1), lambda i: (0, 0)),
        pl.BlockSpec((1, d), lambda i: (0, 0)),
        pl.BlockSpec((d, d), lambda i: (0, 0)),
        pl.BlockSpec((1, d), lambda i: (0, 0)),
        pl.BlockSpec((d, d), lambda i: (0, 0)),
        pl.BlockSpec((1, d), lambda i: (0, 0)),
    ]
    y = pl.pallas_call(
        _tc_nodes_body,
        out_shape=jax.ShapeDtypeStruct((np_, d), jnp.float32),
        grid=(1,),
        in_specs=specs,
        out_specs=pl.BlockSpec((np_, d), lambda i: (0, 0)),
    )(aggs, aggs, hn, d0, d1, b_gcn.reshape(1, d), W_fc1,
      b_fc1.reshape(1, d), W_fc2, b_fc2.reshape(1, d))

    ey = _sc_ey(e, d, nch, ew)(src2, dst2, y)

    dout = W_out.shape[1]
    blk = 4000
    grid = e // blk
    out = pl.pallas_call(
        _tc_edges_body,
        out_shape=jax.ShapeDtypeStruct((e, dout), jnp.float32),
        grid=(grid,),
        in_specs=[
            pl.BlockSpec((blk, d), lambda i: (i, 0)),
            pl.BlockSpec((1, d), lambda i: (0, 0)),
            pl.BlockSpec((d, d), lambda i: (0, 0)),
            pl.BlockSpec((1, d), lambda i: (0, 0)),
            pl.BlockSpec((d, dout), lambda i: (0, 0)),
            pl.BlockSpec((1, dout), lambda i: (0, 0)),
        ],
        out_specs=pl.BlockSpec((blk, dout), lambda i: (i, 0)),
    )(ey, b_fc1.reshape(1, d), W_fc2, b_fc2.reshape(1, d), W_out,
      b_out.reshape(1, dout))
    return out
